# baseline (speedup 1.0000x reference)
import functools

import jax
import jax.numpy as jnp
from jax import lax
from jax.experimental import pallas as pl
from jax.experimental.pallas import tpu as pltpu
from jax.experimental.pallas import tpu_sc as plsc

_N_BINS = 15
_C = 1000
_N = 16384
_A = 12288
_NW = 32
_SPW = _A // _NW
_CH = 16
_NCHUNK = _SPW // _CH
_B = 512


def _sc_body(x_hbm, t_hbm, outc_hbm, outa_hbm,
             xb0, xb1, xb2, tb0, tb1, tb2, oc, oa, dsem):
    wid = lax.axis_index("s") * 2 + lax.axis_index("c")
    wbase = wid * _SPW
    xbufs = (xb0, xb1, xb2)
    tbufs = (tb0, tb1, tb2)

    for h in range(3):
        pltpu.sync_copy(
            t_hbm.at[pl.ds(h + 1, 1), pl.ds(wbase, _SPW)], tbufs[h])

    def _chunk_copies(c, parity):
        return [pltpu.make_async_copy(
            x_hbm.at[pl.ds(wbase + c * _CH, _CH), pl.ds(h, 1)],
            xbufs[h].at[parity], dsem.at[parity]) for h in range(3)]

    for cp in _chunk_copies(0, 0):
        cp.start()

    iota = lax.iota(jnp.int32, 16)
    zi = jnp.zeros((16,), jnp.int32)
    zf = jnp.zeros((16,), jnp.float32)

    def chunk_body(c, carry):
        parity = lax.rem(c, 2)

        @pl.when(c + 1 < _NCHUNK)
        def _prefetch():
            for cp in _chunk_copies(c + 1, 1 - parity):
                cp.start()

        for cp in _chunk_copies(c, parity):
            cp.wait()

        def sample_body(smp, carry2):
            mvs, svs, avs = carry2
            lane = iota == smp
            nmvs, nsvs, navs = [], [], []
            for h in range(3):
                row = xbufs[h].at[parity, smp]
                vms = [jnp.full((16,), -jnp.inf, jnp.float32)
                       for _ in range(4)]
                vidxs = [zi] * 4
                ses = [zf] * 4
                for k in range(62):
                    j = k & 3
                    v = row[0, pl.ds(k * 16, 16)]
                    sel = v > vms[j]
                    vms[j] = jnp.maximum(vms[j], v)
                    vidxs[j] = jnp.where(sel, iota + (k * 16), vidxs[j])
                    ses[j] = ses[j] + jnp.exp(v)
                v = row[0, pl.ds(984, 16)]
                j = 3
                sel = v > vms[j]
                vms[j] = jnp.maximum(vms[j], v)
                vidxs[j] = jnp.where(sel, iota + 984, vidxs[j])
                ses[j] = ses[j] + jnp.where(iota >= 8, jnp.exp(v), 0.0)

                vm, vidx, se = vms[0], vidxs[0], ses[0]
                for j in range(1, 4):
                    gtr = vms[j] > vm
                    tie = vms[j] == vm
                    vidx = jnp.where(
                        gtr, vidxs[j],
                        jnp.where(tie, jnp.minimum(vidx, vidxs[j]), vidx))
                    vm = jnp.maximum(vm, vms[j])
                    se = se + ses[j]

                m = jnp.max(vm)
                srow = jnp.sum(se)
                amax = jnp.min(jnp.where(vm == m, vidx, _C))
                nmvs.append(jnp.where(lane, m, mvs[h]))
                nsvs.append(jnp.where(lane, srow, svs[h]))
                navs.append(jnp.where(lane, amax, avs[h]))
            return (tuple(nmvs), tuple(nsvs), tuple(navs))

        init = ((zf,) * 3, (jnp.ones((16,), jnp.float32),) * 3, (zi,) * 3)
        mvs, svs, avs = lax.fori_loop(0, _CH, sample_body, init,
                                      unroll=False)

        cv = jnp.ones((16,), jnp.float32)
        av = zf
        for h in range(3):
            cv = cv * (jnp.exp(mvs[h]) / svs[h])
            tvh = tbufs[h][0, pl.ds(c * _CH, _CH)]
            av = av + (avs[h] == tvh).astype(jnp.float32)
        oc[pl.ds(c * _CH, _CH)] = cv
        oa[pl.ds(c * _CH, _CH)] = av
        return carry

    lax.fori_loop(0, _NCHUNK, chunk_body, 0, unroll=False)

    pltpu.sync_copy(oc, outc_hbm.at[0, pl.ds(wbase, _SPW)])
    pltpu.sync_copy(oa, outa_hbm.at[0, pl.ds(wbase, _SPW)])


def _bins():
    k = jax.lax.broadcasted_iota(jnp.int32, (1, 16), 1)
    kf = k.astype(jnp.float32)
    lows = jnp.where(k >= _N_BINS, 2.0, kf / _N_BINS)
    highs = jnp.where(k >= _N_BINS, 3.0, (kf + 1.0) / _N_BINS)
    return lows, highs


def _tc_body(x_ref, t_ref, out_ref, acc_ref):
    step = pl.program_id(0)

    @pl.when(step == 0)
    def _init():
        acc_ref[...] = jnp.zeros_like(acc_ref)

    conf = jnp.ones((_B, 1), dtype=jnp.float32)
    acc_row = jnp.zeros((_B, 1), dtype=jnp.float32)
    t = t_ref[...]
    for j in range(3):
        x = x_ref[:, j, :]
        m = jnp.max(x, axis=1, keepdims=True)
        s = jnp.sum(jnp.exp(x - m), axis=1, keepdims=True)
        conf = conf * (1.0 / s)
        iota = jax.lax.broadcasted_iota(jnp.int32, x.shape, 1)
        idx = jnp.min(jnp.where(x == m, iota, _C), axis=1, keepdims=True)
        hit = (idx == t[:, j + 1:j + 2]).astype(jnp.float32)
        acc_row = acc_row + hit

    lows, highs = _bins()
    in_bin = (conf > lows) & (conf <= highs)
    cnt = jnp.sum(in_bin.astype(jnp.float32), axis=0, keepdims=True)
    csum = jnp.sum(jnp.where(in_bin, conf, 0.0), axis=0, keepdims=True)
    asum = jnp.sum(jnp.where(in_bin, acc_row, 0.0), axis=0, keepdims=True)
    acc_ref[0:3, 0:16] += jnp.concatenate([cnt, csum, asum], axis=0)

    @pl.when(step == pl.num_programs(0) - 1)
    def _finish():
        out_ref[...] = acc_ref[0:3, 0:16]


def _combine_body(c_ref, a_ref, p_ref, out_ref, *, n_total):
    conf = c_ref[...]
    acc = a_ref[...]
    part = p_ref[...]
    lows, highs = _bins()
    ece = jnp.zeros((1, 1), jnp.float32)
    for i in range(_N_BINS):
        lo = lows[0, i]
        hi = highs[0, i]
        mask = (conf > lo) & (conf <= hi)
        cnt = jnp.sum(mask.astype(jnp.float32)) + part[0, i]
        cs = jnp.sum(jnp.where(mask, conf, 0.0)) + part[1, i]
        as_ = jnp.sum(jnp.where(mask, acc, 0.0)) + part[2, i]
        safe = jnp.maximum(cnt, 1.0)
        term = jnp.abs(cs / safe - as_ / (safe * 3.0)) * (cnt / n_total)
        term = jnp.where(cnt > 0.0, term, 0.0)
        ece = ece + term * jnp.ones((1, 1), jnp.float32)
    out_ref[...] = ece


def kernel(logits, targets):
    n, hds, c = logits.shape
    assert n == _N and hds == 4 and c == _C
    t32 = targets.astype(jnp.int32)
    ttr = t32.T

    mesh = plsc.VectorSubcoreMesh(core_axis_name="c", subcore_axis_name="s")
    sc_fn = functools.partial(
        pl.kernel,
        mesh=mesh,
        compiler_params=pltpu.CompilerParams(needs_layout_passes=False),
        out_type=(jax.ShapeDtypeStruct((1, _A), jnp.float32),
                  jax.ShapeDtypeStruct((1, _A), jnp.float32)),
        scratch_types=[pltpu.VMEM((2, _CH, 1, _C), jnp.float32)
                       for _ in range(3)]
        + [pltpu.VMEM((1, _SPW), jnp.int32) for _ in range(3)]
        + [pltpu.VMEM((_SPW,), jnp.float32) for _ in range(2)]
        + [pltpu.SemaphoreType.DMA((2,))],
    )(_sc_body)
    conf_v, acc_v = sc_fn(logits, ttr)

    nb = (_N - _A) // _B
    off = _A // _B
    tc_part = pl.pallas_call(
        _tc_body,
        grid=(nb,),
        in_specs=[
            pl.BlockSpec((_B, 4, _C), lambda i: (i + off, 0, 0)),
            pl.BlockSpec((_B, 4), lambda i: (i + off, 0)),
        ],
        out_specs=pl.BlockSpec((3, 16), lambda i: (0, 0)),
        out_shape=jax.ShapeDtypeStruct((3, 16), jnp.float32),
        scratch_shapes=[pltpu.VMEM((8, 128), jnp.float32)],
    )(logits, t32)

    out = pl.pallas_call(
        functools.partial(_combine_body, n_total=float(n)),
        in_specs=[pl.BlockSpec((1, _A), lambda: (0, 0)),
                  pl.BlockSpec((1, _A), lambda: (0, 0)),
                  pl.BlockSpec((3, 16), lambda: (0, 0))],
        out_specs=pl.BlockSpec((1, 1), lambda: (0, 0)),
        out_shape=jax.ShapeDtypeStruct((1, 1), jnp.float32),
    )(conf_v, acc_v, tc_part)
    return out.reshape(1)


# speedup vs baseline: 1.0260x; 1.0260x over previous
import functools

import jax
import jax.numpy as jnp
from jax import lax
from jax.experimental import pallas as pl
from jax.experimental.pallas import tpu as pltpu
from jax.experimental.pallas import tpu_sc as plsc

_N_BINS = 15
_C = 1000
_N = 16384
_A = 8192
_NW = 32
_SPW = _A // _NW
_CH = 16
_NCHUNK = _SPW // _CH
_B = 512


def _sc_body(x_hbm, t_hbm, outc_hbm, outa_hbm,
             xb0, xb1, xb2, tb0, tb1, tb2, oc, oa, dsem):
    wid = lax.axis_index("s") * 2 + lax.axis_index("c")
    wbase = wid * _SPW
    xbufs = (xb0, xb1, xb2)
    tbufs = (tb0, tb1, tb2)

    for h in range(3):
        pltpu.sync_copy(
            t_hbm.at[pl.ds(h + 1, 1), pl.ds(wbase, _SPW)], tbufs[h])

    def _chunk_copies(c, parity):
        return [pltpu.make_async_copy(
            x_hbm.at[pl.ds(wbase + c * _CH, _CH), pl.ds(h, 1)],
            xbufs[h].at[parity], dsem.at[parity]) for h in range(3)]

    for cp in _chunk_copies(0, 0):
        cp.start()

    iota = lax.iota(jnp.int32, 16)
    zi = jnp.zeros((16,), jnp.int32)
    zf = jnp.zeros((16,), jnp.float32)

    def chunk_body(c, carry):
        parity = lax.rem(c, 2)

        @pl.when(c + 1 < _NCHUNK)
        def _prefetch():
            for cp in _chunk_copies(c + 1, 1 - parity):
                cp.start()

        for cp in _chunk_copies(c, parity):
            cp.wait()

        def sample_body(smp, carry2):
            mvs, svs, avs = carry2
            lane = iota == smp
            nmvs, nsvs, navs = [], [], []
            for h in range(3):
                row = xbufs[h].at[parity, smp]
                vms = [jnp.full((16,), -jnp.inf, jnp.float32)
                       for _ in range(4)]
                vidxs = [zi] * 4
                ses = [zf] * 4
                for k in range(62):
                    j = k & 3
                    v = row[0, pl.ds(k * 16, 16)]
                    sel = v > vms[j]
                    vms[j] = jnp.maximum(vms[j], v)
                    vidxs[j] = jnp.where(sel, iota + (k * 16), vidxs[j])
                    ses[j] = ses[j] + jnp.exp(v)
                v = row[0, pl.ds(984, 16)]
                j = 3
                sel = v > vms[j]
                vms[j] = jnp.maximum(vms[j], v)
                vidxs[j] = jnp.where(sel, iota + 984, vidxs[j])
                ses[j] = ses[j] + jnp.where(iota >= 8, jnp.exp(v), 0.0)

                vm, vidx, se = vms[0], vidxs[0], ses[0]
                for j in range(1, 4):
                    gtr = vms[j] > vm
                    tie = vms[j] == vm
                    vidx = jnp.where(
                        gtr, vidxs[j],
                        jnp.where(tie, jnp.minimum(vidx, vidxs[j]), vidx))
                    vm = jnp.maximum(vm, vms[j])
                    se = se + ses[j]

                m = jnp.max(vm)
                srow = jnp.sum(se)
                amax = jnp.min(jnp.where(vm == m, vidx, _C))
                nmvs.append(jnp.where(lane, m, mvs[h]))
                nsvs.append(jnp.where(lane, srow, svs[h]))
                navs.append(jnp.where(lane, amax, avs[h]))
            return (tuple(nmvs), tuple(nsvs), tuple(navs))

        init = ((zf,) * 3, (jnp.ones((16,), jnp.float32),) * 3, (zi,) * 3)
        mvs, svs, avs = lax.fori_loop(0, _CH, sample_body, init,
                                      unroll=False)

        cv = jnp.ones((16,), jnp.float32)
        av = zf
        for h in range(3):
            cv = cv * (jnp.exp(mvs[h]) / svs[h])
            tvh = tbufs[h][0, pl.ds(c * _CH, _CH)]
            av = av + (avs[h] == tvh).astype(jnp.float32)
        oc[pl.ds(c * _CH, _CH)] = cv
        oa[pl.ds(c * _CH, _CH)] = av
        return carry

    lax.fori_loop(0, _NCHUNK, chunk_body, 0, unroll=False)

    pltpu.sync_copy(oc, outc_hbm.at[0, pl.ds(wbase, _SPW)])
    pltpu.sync_copy(oa, outa_hbm.at[0, pl.ds(wbase, _SPW)])


def _bins():
    k = jax.lax.broadcasted_iota(jnp.int32, (1, 16), 1)
    kf = k.astype(jnp.float32)
    lows = jnp.where(k >= _N_BINS, 2.0, kf / _N_BINS)
    highs = jnp.where(k >= _N_BINS, 3.0, (kf + 1.0) / _N_BINS)
    return lows, highs


def _tc_body(x_ref, t_ref, out_ref, acc_ref):
    step = pl.program_id(0)

    @pl.when(step == 0)
    def _init():
        acc_ref[...] = jnp.zeros_like(acc_ref)

    conf = jnp.ones((_B, 1), dtype=jnp.float32)
    acc_row = jnp.zeros((_B, 1), dtype=jnp.float32)
    t = t_ref[...]
    for j in range(3):
        x = x_ref[:, j, :]
        m = jnp.max(x, axis=1, keepdims=True)
        s = jnp.sum(jnp.exp(x - m), axis=1, keepdims=True)
        conf = conf * (1.0 / s)
        iota = jax.lax.broadcasted_iota(jnp.int32, x.shape, 1)
        idx = jnp.min(jnp.where(x == m, iota, _C), axis=1, keepdims=True)
        hit = (idx == t[:, j + 1:j + 2]).astype(jnp.float32)
        acc_row = acc_row + hit

    lows, highs = _bins()
    in_bin = (conf > lows) & (conf <= highs)
    cnt = jnp.sum(in_bin.astype(jnp.float32), axis=0, keepdims=True)
    csum = jnp.sum(jnp.where(in_bin, conf, 0.0), axis=0, keepdims=True)
    asum = jnp.sum(jnp.where(in_bin, acc_row, 0.0), axis=0, keepdims=True)
    acc_ref[0:3, 0:16] += jnp.concatenate([cnt, csum, asum], axis=0)

    @pl.when(step == pl.num_programs(0) - 1)
    def _finish():
        out_ref[...] = acc_ref[0:3, 0:16]


def _combine_body(c_ref, a_ref, p_ref, out_ref, *, n_total):
    conf = c_ref[...]
    acc = a_ref[...]
    part = p_ref[...]
    lows, highs = _bins()
    ece = jnp.zeros((1, 1), jnp.float32)
    for i in range(_N_BINS):
        lo = lows[0, i]
        hi = highs[0, i]
        mask = (conf > lo) & (conf <= hi)
        cnt = jnp.sum(mask.astype(jnp.float32)) + part[0, i]
        cs = jnp.sum(jnp.where(mask, conf, 0.0)) + part[1, i]
        as_ = jnp.sum(jnp.where(mask, acc, 0.0)) + part[2, i]
        safe = jnp.maximum(cnt, 1.0)
        term = jnp.abs(cs / safe - as_ / (safe * 3.0)) * (cnt / n_total)
        term = jnp.where(cnt > 0.0, term, 0.0)
        ece = ece + term * jnp.ones((1, 1), jnp.float32)
    out_ref[...] = ece


def kernel(logits, targets):
    n, hds, c = logits.shape
    assert n == _N and hds == 4 and c == _C
    t32 = targets.astype(jnp.int32)
    ttr = t32.T

    mesh = plsc.VectorSubcoreMesh(core_axis_name="c", subcore_axis_name="s")
    sc_fn = functools.partial(
        pl.kernel,
        mesh=mesh,
        compiler_params=pltpu.CompilerParams(needs_layout_passes=False),
        out_type=(jax.ShapeDtypeStruct((1, _A), jnp.float32),
                  jax.ShapeDtypeStruct((1, _A), jnp.float32)),
        scratch_types=[pltpu.VMEM((2, _CH, 1, _C), jnp.float32)
                       for _ in range(3)]
        + [pltpu.VMEM((1, _SPW), jnp.int32) for _ in range(3)]
        + [pltpu.VMEM((_SPW,), jnp.float32) for _ in range(2)]
        + [pltpu.SemaphoreType.DMA((2,))],
    )(_sc_body)
    conf_v, acc_v = sc_fn(logits, ttr)

    nb = (_N - _A) // _B
    off = _A // _B
    tc_part = pl.pallas_call(
        _tc_body,
        grid=(nb,),
        in_specs=[
            pl.BlockSpec((_B, 4, _C), lambda i: (i + off, 0, 0)),
            pl.BlockSpec((_B, 4), lambda i: (i + off, 0)),
        ],
        out_specs=pl.BlockSpec((3, 16), lambda i: (0, 0)),
        out_shape=jax.ShapeDtypeStruct((3, 16), jnp.float32),
        scratch_shapes=[pltpu.VMEM((8, 128), jnp.float32)],
    )(logits, t32)

    out = pl.pallas_call(
        functools.partial(_combine_body, n_total=float(n)),
        in_specs=[pl.BlockSpec((1, _A), lambda: (0, 0)),
                  pl.BlockSpec((1, _A), lambda: (0, 0)),
                  pl.BlockSpec((3, 16), lambda: (0, 0))],
        out_specs=pl.BlockSpec((1, 1), lambda: (0, 0)),
        out_shape=jax.ShapeDtypeStruct((1, 1), jnp.float32),
    )(conf_v, acc_v, tc_part)
    return out.reshape(1)
